# baseline (speedup 1.0000x reference)
import jax
import jax.numpy as jnp
from jax import lax
from jax.experimental import pallas as pl


N_OBJ = 5000
NUM_OBJ_CLS = 151
N_REL = 20000
REL_DIM = 4096
NUM_REL_CLS = 51

BM = 1024
GRID = (N_REL + BM - 1) // BM
BOBJ = 256
N_OBJ_PAD = GRID * BOBJ


def _fused_body(vr_ref, w_ref, b_ref, objt_ref, out_ref, pred_ref):
    acc = lax.dot_general(
        w_ref[...], vr_ref[...],
        (((1,), (1,)), ((), ())),
        preferred_element_type=jnp.float32,
    )
    out_ref[...] = acc + b_ref[...]
    am = jnp.argmax(objt_ref[1:, :], axis=0).astype(jnp.int32) + 1
    pred_ref[...] = am.reshape(pred_ref.shape)


@jax.jit
def kernel(obj_logits, vr, W, b):
    b_col = b.reshape(NUM_REL_CLS, 1)
    obj_t = obj_logits.T

    rel_t, preds = pl.pallas_call(
        _fused_body,
        grid=(GRID,),
        in_specs=[
            pl.BlockSpec((BM, REL_DIM), lambda i: (i, 0)),
            pl.BlockSpec((NUM_REL_CLS, REL_DIM), lambda i: (0, 0)),
            pl.BlockSpec((NUM_REL_CLS, 1), lambda i: (0, 0)),
            pl.BlockSpec((NUM_OBJ_CLS, BOBJ), lambda i: (0, i)),
        ],
        out_specs=[
            pl.BlockSpec((NUM_REL_CLS, BM), lambda i: (0, i)),
            pl.BlockSpec((1, BOBJ), lambda i: (0, i)),
        ],
        out_shape=[
            jax.ShapeDtypeStruct((NUM_REL_CLS, N_REL), jnp.float32),
            jax.ShapeDtypeStruct((1, N_OBJ_PAD), jnp.int32),
        ],
    )(vr, W, b_col, obj_t)

    rel_dists = rel_t.T
    obj_preds = preds[0, :N_OBJ]
    return obj_logits, obj_preds, rel_dists
